# baseline (speedup 1.0000x reference)
import math

import jax
import jax.numpy as jnp
from jax.experimental import pallas as pl
from jax.experimental.pallas import tpu as pltpu

D = 64
V = 8192
B = 16
T = 32
SLOTS = 32
MAX_REC = 8
PONDER_COST = 0.01
VCHUNK = 512
NCHUNK = V // VCHUNK

_ACT, _VQ, _ENT, _POND, _PH = 0, 1, 2, 3, 4


def _xsum(x):
    w = x.shape[-1]
    p = x[..., 0:8]
    for m in range(1, w // 8):
        p = p + x[..., 8 * m:8 * m + 8]
    q = p[..., 0:4] + p[..., 4:8]
    r = q[..., 0:2] + q[..., 2:4]
    return r[..., 0:1] + r[..., 1:2]


def _xsum_slots(x):
    p = x[:, 0:8] + x[:, 8:16]
    p = p + x[:, 16:24]
    p = p + x[:, 24:32]
    q = p[:, 0:4] + p[:, 4:8]
    r = q[:, 0:2] + q[:, 2:4]
    s = r[:, 0:1] + r[:, 1:2]
    return s[:, 0]


def _body(x_seq_ref, scal_ref, enc_ref, vqhi_ref, vqlo_ref, vqlo2_ref,
          esq_ref,
          qWr_ref, qWi_ref, kWr_ref, kWi_ref, vWr_ref, vWi_ref,
          qbr_ref, qbi_ref, kbr_ref, kbi_ref, vbr_ref, vbi_ref,
          arbW_ref, arbb_ref, gateW_ref, lng_ref, lnb_ref,
          decW_ref, decb_ref,
          logits_ref, stats_ref, idx_ref,
          memc_ref, flat_ref, xb_ref, gw_ref, acc_ref, fin_ref, pang_ref):
    alpha = scal_ref[0, 0]
    hbias = scal_ref[0, 1]
    gate_b = scal_ref[0, 2]

    memc_ref[...] = jnp.zeros_like(memc_ref)
    idx_ref[...] = jnp.zeros_like(idx_ref)
    gw_ref[...] = jnp.zeros_like(gw_ref)

    arbW = arbW_ref[...]
    arbb = arbb_ref[...]
    gateW = gateW_ref[...]
    lng = lng_ref[...]
    lnb = lnb_ref[...]

    def ln_half(h, g, b):
        m = _xsum(h) * (1.0 / D)
        xc = h - m
        var = _xsum(xc * xc) * (1.0 / D)
        return xc * jax.lax.rsqrt(var + 1e-5) * g + b

    def clin(cr, ci, wr_ref, wi_ref, br_ref, bi_ref):
        ar = jnp.dot(cr, wr_ref[...], preferred_element_type=jnp.float32)
        bi_r = jnp.dot(ci, wi_ref[...], preferred_element_type=jnp.float32)
        ai = jnp.dot(ci, wr_ref[...], preferred_element_type=jnp.float32)
        br_i = jnp.dot(cr, wi_ref[...], preferred_element_type=jnp.float32)
        br = br_ref[...]
        bb = bi_ref[...]
        return (ar + br) - (bi_r + bb), (ai + br) + (br_i + bb)

    def vq_search(z):
        zsq = _xsum(z * z)
        big = jnp.float32(1e30)
        zb = z.astype(jnp.bfloat16)

        def chunk(c, carry):
            best, bidx, zq = carry
            hi = vqhi_ref[pl.ds(c * VCHUNK, VCHUNK), :]
            s = jax.lax.dot_general(zb, hi, (((1,), (1,)), ((), ())),
                                    preferred_element_type=jnp.float32)
            d2 = (zsq - 2.0 * s) + esq_ref[:, pl.ds(c * VCHUNK, VCHUNK)]
            lmin = jnp.min(d2, axis=-1, keepdims=True)
            lio = (jax.lax.broadcasted_iota(jnp.int32, (B, VCHUNK), 1)
                   + c * VCHUNK)
            lidx = jnp.min(jnp.where(d2 == lmin, lio, jnp.int32(2 ** 30)),
                           axis=-1, keepdims=True)
            oh = (lio == lidx).astype(jnp.bfloat16)
            lo = vqlo_ref[pl.ds(c * VCHUNK, VCHUNK), :]
            lo2 = vqlo2_ref[pl.ds(c * VCHUNK, VCHUNK), :]
            zql = (jax.lax.dot_general(oh, hi, (((1,), (0,)), ((), ())),
                                       preferred_element_type=jnp.float32)
                   + jax.lax.dot_general(oh, lo, (((1,), (0,)), ((), ())),
                                         preferred_element_type=jnp.float32)
                   + jax.lax.dot_general(oh, lo2, (((1,), (0,)), ((), ())),
                                         preferred_element_type=jnp.float32))
            take = lmin < best
            best = jnp.where(take, lmin, best)
            bidx = jnp.where(take, lidx, bidx)
            zq = jnp.where(take, zql, zq)
            return best, bidx, zq

        z0 = z[:, 0:1]
        init = (jnp.where(jnp.abs(z0) > big, z0, big),
                (z0 > big).astype(jnp.int32),
                jnp.where(jnp.abs(z) > big, z, 0.0))
        _, bidx, zq = jax.lax.fori_loop(0, NCHUNK, chunk, init)
        return bidx, zq

    def entropy(bidx):
        def entchunk(c, cb):
            lio = (jax.lax.broadcasted_iota(jnp.int32, (B, VCHUNK), 1)
                   + c * VCHUNK)
            ohc = (lio == bidx).astype(jnp.float32)
            cnt = jnp.sum(ohc, axis=0, keepdims=True)
            return cb + jnp.sum(ohc * cnt, axis=-1, keepdims=True)

        c_b = jax.lax.fori_loop(0, NCHUNK, entchunk,
                                (bidx < 0).astype(jnp.float32))
        logs = jnp.log(c_b * (1.0 / B) + 1e-10)
        return -jnp.sum(logs, axis=0, keepdims=True) * (1.0 / B)

    def rec_step(st):
        it, _ = st
        gwc = gw_ref[...]
        active = acc_ref[:, _ACT:_ACT + 1]
        pang = pang_ref[...]

        cr = ln_half(gwc[:, :D], lng[:, :D], lnb[:, :D])
        ci = ln_half(gwc[:, D:], lng[:, D:], lnb[:, D:])
        z = jnp.concatenate([cr, ci], axis=-1)
        bidx, zq = vq_search(z)
        zqst = z + (zq - z)
        dqz = zq - z
        a = jnp.sum(dqz * dqz, axis=-1, keepdims=True) * (1.0 / (2 * D))
        vq_loss = a + 0.25 * a
        ent = entropy(bidx)

        qr, qi = clin(cr, ci, qWr_ref, qWi_ref, qbr_ref, qbi_ref)
        kr, ki = clin(cr, ci, kWr_ref, kWi_ref, kbr_ref, kbi_ref)
        vr, vi = clin(cr, ci, vWr_ref, vWi_ref, vbr_ref, vbi_ref)
        gate = jax.nn.sigmoid(_xsum(qr * kr + qi * ki))
        g_r = vr * gate
        g_i = vi * gate

        mem = memc_ref[...]
        sim = (_xsum(mem[:, :, :D] * cr[:, None, :]
                     + mem[:, :, D:] * ci[:, None, :]))[:, :, 0]
        mx = jnp.max(sim, axis=-1, keepdims=True)
        e = jnp.exp(sim - mx)
        attn = e / _xsum(e)
        mcat = _xsum_slots(mem * attn[:, :, None])

        ga = jax.nn.softmax(
            jnp.dot(z, arbW, preferred_element_type=jnp.float32) + arbb,
            axis=-1)
        ga0 = ga[:, 0:1]
        ga1 = ga[:, 1:2]
        ga2 = ga[:, 2:3]
        up_r = ga0 * g_r + ga1 * mcat[:, :D] + ga2 * zqst[:, :D]
        up_i = ga0 * g_i + ga1 * mcat[:, D:] + ga2 * zqst[:, D:]
        cand_r = 0.6 * cr + 0.4 * up_r
        cand_i = 0.6 * ci + 0.4 * up_i

        ang = jnp.arctan2(cand_i, cand_r)
        diff = jnp.abs(ang - pang)
        diff = jnp.minimum(diff, 2.0 * math.pi - diff)
        acc_ref[:, _PH:_PH + 1] += active * (
            jnp.sum(diff, axis=-1, keepdims=True) * (1.0 / D))
        pang_ref[...] = ang

        stop = (hbias - vq_loss > 0.0).astype(jnp.float32)
        acc_ref[:, _POND:_POND + 1] += active * PONDER_COST
        maskf = active > 0.5
        acc_ref[:, _VQ:_VQ + 1] = jnp.where(
            maskf, vq_loss, acc_ref[:, _VQ:_VQ + 1])
        acc_ref[:, _ENT:_ENT + 1] = jnp.where(
            maskf, jnp.broadcast_to(ent, (B, 1)), acc_ref[:, _ENT:_ENT + 1])
        fin_ref[...] = jnp.where(maskf, bidx, fin_ref[...])
        gw_ref[...] = jnp.where(
            maskf, jnp.concatenate([cand_r, cand_i], axis=-1), gwc)
        new_active = active * (1.0 - stop)
        acc_ref[:, _ACT:_ACT + 1] = new_active
        return it + 1, jnp.max(new_active) > 0.5

    def rec_cond(st):
        it, go = st
        return jnp.logical_and(it < MAX_REC, go)

    lane32 = jax.lax.broadcasted_iota(jnp.int32, (B, T), 1)
    acc_init = (jax.lax.broadcasted_iota(jnp.int32, (B, 8), 1)
                == _ACT).astype(jnp.float32)

    def tstep(t, carry):
        s0, s1, s2, s3 = carry
        for b in range(B):
            xb_ref[pl.ds(b, 1), :] = enc_ref[pl.ds(x_seq_ref[b, t], 1), :]
        xc = xb_ref[...]
        gwc = alpha * gw_ref[...] + (1.0 - alpha) * xc
        gw_ref[...] = gwc

        pang_ref[...] = jnp.arctan2(gwc[:, D:], gwc[:, :D])
        acc_ref[...] = acc_init
        fin_ref[...] = (gwc[:, 0:1] > jnp.float32(1e30)).astype(jnp.int32)
        jax.lax.while_loop(rec_cond, rec_step, (jnp.int32(0), True))

        gwc = gw_ref[...]
        wg = jax.nn.sigmoid(_xsum(gwc * gateW) + gate_b)
        mem = memc_ref[...]
        last = mem[:, SLOTS - 1, :]
        head0 = wg * gwc + (1.0 - wg) * last
        memc_ref[...] = jnp.concatenate(
            [head0[:, None, :], mem[:, :SLOTS - 1, :]], axis=1)

        flat_ref[pl.ds(pl.multiple_of(t * B, B), B), :] = gwc
        idx_ref[...] = jnp.where(
            lane32 == t, jnp.broadcast_to(fin_ref[...], (B, T)), idx_ref[...])

        s0 = s0 + jnp.sum(acc_ref[:, _VQ:_VQ + 1], axis=0, keepdims=True)
        s1 = s1 + jnp.sum(acc_ref[:, _ENT:_ENT + 1], axis=0, keepdims=True)
        s2 = s2 + jnp.sum(acc_ref[:, _POND:_POND + 1], axis=0, keepdims=True)
        s3 = s3 + jnp.sum(acc_ref[:, _PH:_PH + 1], axis=0, keepdims=True)
        return s0, s1, s2, s3

    z11 = jnp.zeros((1, 1), jnp.float32)
    s0, s1, s2, s3 = jax.lax.fori_loop(0, T, tstep, (z11, z11, z11, z11))

    stats_ref[...] = (jnp.concatenate([s0, s1, s2, s3], axis=1)
                      * (1.0 / (B * T)))
    fl = flat_ref[...].astype(jnp.bfloat16)
    logits_ref[...] = jax.lax.dot_general(
        fl, decW_ref[...], (((1,), (1,)), ((), ())),
        preferred_element_type=jnp.float32) + decb_ref[...]


def kernel(x_seq, params):
    p = params
    f32 = jnp.float32
    esq = (p['vq_emb'] ** 2).sum(-1)[None, :]
    vq_hi = p['vq_emb'].astype(jnp.bfloat16)
    r1 = p['vq_emb'] - vq_hi.astype(f32)
    vq_lo = r1.astype(jnp.bfloat16)
    vq_lo2 = (r1 - vq_lo.astype(f32)).astype(jnp.bfloat16)
    scal = jnp.stack([jax.nn.sigmoid(p['input_gate']),
                      jax.nn.softplus(p['halt_bias']),
                      p['gate_b'][0], jnp.float32(0.0)])[None].astype(f32)
    lngc = jnp.concatenate([p['ln_r_g'], p['ln_i_g']])[None]
    lnbc = jnp.concatenate([p['ln_r_b'], p['ln_i_b']])[None]

    logits_tm, stats, idx = pl.pallas_call(
        _body,
        out_shape=(
            jax.ShapeDtypeStruct((B * T, V), f32),
            jax.ShapeDtypeStruct((1, 4), f32),
            jax.ShapeDtypeStruct((B, T), jnp.int32),
        ),
        in_specs=[
            pl.BlockSpec(memory_space=pltpu.SMEM),
            pl.BlockSpec(memory_space=pltpu.SMEM),
        ] + [pl.BlockSpec()] * 24,
        scratch_shapes=[
            pltpu.VMEM((B, SLOTS, 2 * D), f32),
            pltpu.VMEM((B * T, 2 * D), f32),
            pltpu.VMEM((B, 2 * D), f32),
            pltpu.VMEM((B, 2 * D), f32),
            pltpu.VMEM((B, 8), f32),
            pltpu.VMEM((B, 1), jnp.int32),
            pltpu.VMEM((B, D), f32),
        ],
        compiler_params=pltpu.CompilerParams(
            vmem_limit_bytes=60 * 1024 * 1024),
    )(x_seq, scal, p['enc'], vq_hi, vq_lo, vq_lo2, esq,
      p['qW_r'].T, p['qW_i'].T, p['kW_r'].T, p['kW_i'].T,
      p['vW_r'].T, p['vW_i'].T,
      p['qb_r'][None], p['qb_i'][None], p['kb_r'][None], p['kb_i'][None],
      p['vb_r'][None], p['vb_i'][None],
      p['arb_W'].T, p['arb_b'][None], p['gate_W'],
      lngc, lnbc, p['dec_W'].astype(jnp.bfloat16), p['dec_b'][None])

    logits = logits_tm.reshape(T, B, V).transpose(1, 0, 2)
    return logits, stats.reshape(4), idx
